# Initial kernel scaffold; baseline (speedup 1.0000x reference)
import jax
import jax.numpy as jnp
from jax.experimental import pallas as pl


def kernel(samples, W_s, b_s, W_t, b_t):
    raise NotImplementedError("write your pallas kernel here")


# speedup vs baseline: 20.5492x; 20.5492x over previous
import jax
import jax.numpy as jnp
from jax.experimental import pallas as pl
from jax.experimental.pallas import tpu as pltpu

_DIM = 128
_HALF = 64
_BLOCK = 1024


def _coupling_kernel(x_ref, w_ref, b_ref, out_ref, jac_ref):
    x = x_ref[:, :]
    z1 = x[:, :_HALF]
    z2 = x[:, _HALF:]
    st = jnp.dot(z2, w_ref[:, :], preferred_element_type=jnp.float32) + b_ref[0, :]
    s = jnp.tanh(st[:, :_HALF])
    t = st[:, _HALF:]
    x1 = z1 * jnp.exp(s) + t
    out_ref[:, :] = jnp.concatenate([x1, z2], axis=1)
    jac_ref[:] = jnp.sum(s, axis=1)


def kernel(samples, W_s, b_s, W_t, b_t):
    n = samples.shape[0]
    w = jnp.concatenate([W_s, W_t], axis=1)
    b = jnp.concatenate([b_s, b_t]).reshape(1, _DIM)
    grid = n // _BLOCK
    res, jac = pl.pallas_call(
        _coupling_kernel,
        grid=(grid,),
        in_specs=[
            pl.BlockSpec((_BLOCK, _DIM), lambda i: (i, 0)),
            pl.BlockSpec((_HALF, _DIM), lambda i: (0, 0)),
            pl.BlockSpec((1, _DIM), lambda i: (0, 0)),
        ],
        out_specs=[
            pl.BlockSpec((_BLOCK, _DIM), lambda i: (i, 0)),
            pl.BlockSpec((_BLOCK,), lambda i: (i,)),
        ],
        out_shape=[
            jax.ShapeDtypeStruct((n, _DIM), jnp.float32),
            jax.ShapeDtypeStruct((n,), jnp.float32),
        ],
        compiler_params=pltpu.CompilerParams(
            dimension_semantics=("parallel",),
        ),
    )(samples, w, b)
    return (res, jac)
